# baseline (speedup 1.0000x reference)
import functools

import jax
import jax.numpy as jnp
from jax import lax
from jax.experimental import pallas as pl
from jax.experimental.pallas import tpu as pltpu
from jax.experimental.pallas import tpu_sc as plsc

BATCH = 16384
DIM = 128
HALF = 64
LANES = 16

NUM_CORES = 2
NUM_SUBCORES = 16
NUM_WORKERS = NUM_CORES * NUM_SUBCORES
ROWS_PER_WORKER = BATCH // NUM_WORKERS
CHUNK = 128
NCHUNK = ROWS_PER_WORKER // CHUNK
GROUPS = CHUNK // LANES
NBUF = 2


def _perm(a, idx):
  dnums = lax.GatherDimensionNumbers(
      offset_dims=(), collapsed_slice_dims=(0,), start_index_map=(0,))
  return lax.gather(a, idx[:, None], dimension_numbers=dnums,
                    slice_sizes=(1,),
                    mode=lax.GatherScatterMode.PROMISE_IN_BOUNDS)


def _sc_body(h_hbm, r_hbm, t_hbm, rel_hbm, out_hbm,
             idx_v, h_v, t_v, rel_v, score_v, acc_v, rel_sh,
             sem_idx, sem_h, sem_t, sem_rel, sem_out, sem_st):
  cid = lax.axis_index("c")
  sid = lax.axis_index("s")
  wid = cid * NUM_SUBCORES + sid
  lanes = lax.iota(jnp.int32, LANES)

  idx_cp = pltpu.async_copy(
      r_hbm.at[pl.ds(wid * ROWS_PER_WORKER, ROWS_PER_WORKER)], idx_v, sem_idx)

  @pl.when(sid == 0)
  def _stage():
    pltpu.async_copy(rel_hbm, rel_sh, sem_st).wait()
  idx_cp.wait()
  plsc.subcore_barrier()

  def issue(c, s):
    base = wid * ROWS_PER_WORKER + c * CHUNK
    return (pltpu.async_copy(h_hbm.at[pl.ds(base, CHUNK)], h_v.at[s], sem_h.at[s]),
            pltpu.async_copy(t_hbm.at[pl.ds(base, CHUNK)], t_v.at[s], sem_t.at[s]),
            pltpu.async_copy(rel_sh.at[idx_v.at[pl.ds(c * CHUNK, CHUNK)]], rel_v.at[s], sem_rel.at[s]))

  def compute(c, s):
    def merge(a, b, bit):
      perm = lanes ^ bit
      a2 = a + _perm(a, perm)
      b2 = b + _perm(b, perm)
      return jnp.where((lanes & bit) == 0, a2, b2)

    @plsc.parallel_loop(0, CHUNK // 2, unroll=1)
    def row_pass(i):
      accs = []
      for half_r in range(2):
        r = 2 * i + half_r
        acc = None
        for j in range(HALF // LANES):
          hr = h_v[s, r, pl.ds(j * LANES, LANES)]
          hi = h_v[s, r, pl.ds(HALF + j * LANES, LANES)]
          tr = t_v[s, r, pl.ds(j * LANES, LANES)]
          ti = t_v[s, r, pl.ds(HALF + j * LANES, LANES)]
          rr = rel_v[s, r, pl.ds(j * LANES, LANES)]
          ri = rel_v[s, r, pl.ds(HALF + j * LANES, LANES)]
          term = rr * (hr * tr + hi * ti) + ri * (hr * ti - hi * tr)
          acc = term if acc is None else acc + term
        accs.append(acc)
      acc_v[i] = merge(accs[0], accs[1], 1)

    @plsc.parallel_loop(0, GROUPS, unroll=1)
    def group(g):
      stack = []
      for k in range(LANES // 2):
        node = (1, acc_v[g * (LANES // 2) + k])
        while stack and stack[-1][0] == node[0]:
          lvl, left = stack.pop()
          node = (lvl + 1, merge(left, node[1], 1 << lvl))
        stack.append(node)
      score_v[s, pl.ds(g * LANES, LANES)] = stack[0][1]

    base = wid * ROWS_PER_WORKER + c * CHUNK
    return pltpu.async_copy(score_v.at[s], out_hbm.at[pl.ds(base, CHUNK)],
                            sem_out.at[s])

  pending = issue(0, 0)
  out_cp = [None] * NCHUNK
  for c in range(NCHUNK):
    s = c % NBUF
    nxt = issue(c + 1, (c + 1) % NBUF) if c + 1 < NCHUNK else None
    for cp in pending:
      cp.wait()
    if c >= NBUF and out_cp[c - NBUF] is not None:
      out_cp[c - NBUF].wait()
    out_cp[c] = compute(c, s)
    pending = nxt
  for c in range(NCHUNK - NBUF, NCHUNK):
    out_cp[c].wait()


_sc_kernel = functools.partial(
    pl.kernel,
    out_type=jax.ShapeDtypeStruct((BATCH,), jnp.float32),
    mesh=plsc.VectorSubcoreMesh(core_axis_name="c", subcore_axis_name="s"),
    scratch_types=[
        pltpu.VMEM((ROWS_PER_WORKER,), jnp.int32),
        pltpu.VMEM((NBUF, CHUNK, DIM), jnp.float32),
        pltpu.VMEM((NBUF, CHUNK, DIM), jnp.float32),
        pltpu.VMEM((NBUF, CHUNK, DIM), jnp.float32),
        pltpu.VMEM((NBUF, CHUNK), jnp.float32),
        pltpu.VMEM((CHUNK // 2, LANES), jnp.float32),
        pltpu.VMEM_SHARED((1000, DIM), jnp.float32),
        pltpu.SemaphoreType.DMA,
        pltpu.SemaphoreType.DMA((NBUF,)),
        pltpu.SemaphoreType.DMA((NBUF,)),
        pltpu.SemaphoreType.DMA((NBUF,)),
        pltpu.SemaphoreType.DMA((NBUF,)),
        pltpu.SemaphoreType.DMA,
    ],
)(_sc_body)


@jax.jit
def kernel(h_emb, r_idx, t_emb, re_rel, im_rel):
  rel_cat = jnp.concatenate([re_rel, im_rel], axis=1)
  return _sc_kernel(h_emb, r_idx.astype(jnp.int32), t_emb, rel_cat)


# speedup vs baseline: 1.0085x; 1.0085x over previous
import functools

import jax
import jax.numpy as jnp
from jax import lax
from jax.experimental import pallas as pl
from jax.experimental.pallas import tpu as pltpu
from jax.experimental.pallas import tpu_sc as plsc

BATCH = 16384
DIM = 128
HALF = 64
LANES = 16

NUM_CORES = 2
NUM_SUBCORES = 16
NUM_WORKERS = NUM_CORES * NUM_SUBCORES
ROWS_PER_WORKER = BATCH // NUM_WORKERS
CHUNK = 128
NCHUNK = ROWS_PER_WORKER // CHUNK
GROUPS = CHUNK // LANES
NBUF = 2


def _perm(a, idx):
  dnums = lax.GatherDimensionNumbers(
      offset_dims=(), collapsed_slice_dims=(0,), start_index_map=(0,))
  return lax.gather(a, idx[:, None], dimension_numbers=dnums,
                    slice_sizes=(1,),
                    mode=lax.GatherScatterMode.PROMISE_IN_BOUNDS)


def _sc_body(h_hbm, r_hbm, t_hbm, rel_hbm, out_hbm,
             idx_v, h_v, t_v, rel_v, score_v, acc_v, rel_sh,
             sem_idx, sem_h, sem_t, sem_rel, sem_out, sem_st):
  cid = lax.axis_index("c")
  sid = lax.axis_index("s")
  wid = cid * NUM_SUBCORES + sid
  lanes = lax.iota(jnp.int32, LANES)

  idx_cp = pltpu.async_copy(
      r_hbm.at[pl.ds(wid * ROWS_PER_WORKER, ROWS_PER_WORKER)], idx_v, sem_idx)

  @pl.when(sid == 0)
  def _stage():
    pltpu.async_copy(rel_hbm, rel_sh, sem_st).wait()
  idx_cp.wait()
  plsc.subcore_barrier()

  def issue(c, s):
    base = wid * ROWS_PER_WORKER + c * CHUNK
    return (pltpu.async_copy(h_hbm.at[pl.ds(base, CHUNK)], h_v.at[s], sem_h.at[s]),
            pltpu.async_copy(t_hbm.at[pl.ds(base, CHUNK)], t_v.at[s], sem_t.at[s]),
            pltpu.async_copy(rel_sh.at[idx_v.at[pl.ds(c * CHUNK, CHUNK)]], rel_v.at[s], sem_rel.at[s]))

  def compute(c, s):
    def merge(a, b, bit):
      perm = lanes ^ bit
      a2 = a + _perm(a, perm)
      b2 = b + _perm(b, perm)
      return jnp.where((lanes & bit) == 0, a2, b2)

    @plsc.parallel_loop(0, CHUNK, unroll=1)
    def row_pass(r):
      acc = None
      for j in range(HALF // LANES):
        hr = h_v[s, r, pl.ds(j * LANES, LANES)]
        hi = h_v[s, r, pl.ds(HALF + j * LANES, LANES)]
        tr = t_v[s, r, pl.ds(j * LANES, LANES)]
        ti = t_v[s, r, pl.ds(HALF + j * LANES, LANES)]
        rr = rel_v[s, r, pl.ds(j * LANES, LANES)]
        ri = rel_v[s, r, pl.ds(HALF + j * LANES, LANES)]
        term = rr * (hr * tr + hi * ti) + ri * (hr * ti - hi * tr)
        acc = term if acc is None else acc + term
      acc_v[r] = acc

    @plsc.parallel_loop(0, GROUPS, unroll=1)
    def group(g):
      stack = []
      for k in range(LANES):
        node = (0, acc_v[g * LANES + k])
        while stack and stack[-1][0] == node[0]:
          lvl, left = stack.pop()
          node = (lvl + 1, merge(left, node[1], 1 << lvl))
        stack.append(node)
      score_v[s, pl.ds(g * LANES, LANES)] = stack[0][1]

    base = wid * ROWS_PER_WORKER + c * CHUNK
    return pltpu.async_copy(score_v.at[s], out_hbm.at[pl.ds(base, CHUNK)],
                            sem_out.at[s])

  pending = issue(0, 0)
  out_cp = [None] * NCHUNK
  for c in range(NCHUNK):
    s = c % NBUF
    nxt = issue(c + 1, (c + 1) % NBUF) if c + 1 < NCHUNK else None
    for cp in pending:
      cp.wait()
    if c >= NBUF and out_cp[c - NBUF] is not None:
      out_cp[c - NBUF].wait()
    out_cp[c] = compute(c, s)
    pending = nxt
  for c in range(NCHUNK - NBUF, NCHUNK):
    out_cp[c].wait()


_sc_kernel = functools.partial(
    pl.kernel,
    out_type=jax.ShapeDtypeStruct((BATCH,), jnp.float32),
    mesh=plsc.VectorSubcoreMesh(core_axis_name="c", subcore_axis_name="s"),
    scratch_types=[
        pltpu.VMEM((ROWS_PER_WORKER,), jnp.int32),
        pltpu.VMEM((NBUF, CHUNK, DIM), jnp.float32),
        pltpu.VMEM((NBUF, CHUNK, DIM), jnp.float32),
        pltpu.VMEM((NBUF, CHUNK, DIM), jnp.float32),
        pltpu.VMEM((NBUF, CHUNK), jnp.float32),
        pltpu.VMEM((CHUNK, LANES), jnp.float32),
        pltpu.VMEM_SHARED((1000, DIM), jnp.float32),
        pltpu.SemaphoreType.DMA,
        pltpu.SemaphoreType.DMA((NBUF,)),
        pltpu.SemaphoreType.DMA((NBUF,)),
        pltpu.SemaphoreType.DMA((NBUF,)),
        pltpu.SemaphoreType.DMA((NBUF,)),
        pltpu.SemaphoreType.DMA,
    ],
)(_sc_body)


@jax.jit
def kernel(h_emb, r_idx, t_emb, re_rel, im_rel):
  rel_cat = jnp.concatenate([re_rel, im_rel], axis=1)
  return _sc_kernel(h_emb, r_idx.astype(jnp.int32), t_emb, rel_cat)
